# Initial kernel scaffold; baseline (speedup 1.0000x reference)
import jax
import jax.numpy as jnp
from jax.experimental import pallas as pl


def kernel(inputs, atom_table, num_table):
    raise NotImplementedError("write your pallas kernel here")


# speedup vs baseline: 12.0815x; 12.0815x over previous
import functools

import jax
import jax.numpy as jnp
from jax import lax
from jax.experimental import pallas as pl
from jax.experimental.pallas import tpu as pltpu
from jax.experimental.pallas import tpu_sc as plsc

DIM = 48
ND = 6
NSLOT = 8
IN_W = 41
OUT_W = 80
L = 16
NC, NS = 2, 16
NW = NC * NS

B, N = 1024, 512
P = B * N
PW = P // NW
CHUNK = 512
GRP = CHUNK // L
NCHUNK = PW // CHUNK


def _sc_kernel(in_hbm, atom_hbm, num_hbm, out_hbm, in_v, out_v, atom_v, num_v):
    wid = lax.axis_index("s") * NC + lax.axis_index("c")

    pltpu.sync_copy(atom_hbm, atom_v)
    pltpu.sync_copy(num_hbm, num_v)

    lane = lax.iota(jnp.int32, 16)
    lane_in = lane * IN_W
    lane_out = lane * OUT_W

    def chunk_body(k, carry):
        base = wid * PW + k * CHUNK
        pltpu.sync_copy(in_hbm.at[pl.ds(base * IN_W, CHUNK * IN_W)], in_v)

        def grp_body(g, c2):
            ioff = lane_in + g * (L * IN_W)
            ooff = lane_out + g * (L * OUT_W)
            names = plsc.load_gather(in_v, [ioff])
            aaddr = names.astype(jnp.int32) * DIM
            naddr = []
            for j in range(NSLOT):
                nj = plsc.load_gather(in_v, [ioff + (33 + j)])
                naddr.append(nj.astype(jnp.int32) * ND)
            for c in range(DIM):
                va = plsc.load_gather(atom_v, [aaddr + c])
                vn = plsc.load_gather(num_v, [naddr[c // ND] + (c % ND)])
                plsc.store_scatter(out_v, [ooff + c], va + vn)
            for c in range(29):
                v = plsc.load_gather(in_v, [ioff + (4 + c)])
                plsc.store_scatter(out_v, [ooff + (DIM + c)], v)
            for c in range(3):
                v = plsc.load_gather(in_v, [ioff + (1 + c)])
                plsc.store_scatter(out_v, [ooff + (77 + c)], v)
            return c2

        lax.fori_loop(0, GRP, grp_body, 0)
        pltpu.sync_copy(out_v, out_hbm.at[pl.ds(base * OUT_W, CHUNK * OUT_W)])
        return carry

    lax.fori_loop(0, NCHUNK, chunk_body, 0)


@jax.jit
def kernel(inputs, atom_table, num_table):
    mesh = plsc.VectorSubcoreMesh(
        core_axis_name="c", subcore_axis_name="s", num_cores=NC, num_subcores=NS
    )
    run = functools.partial(
        pl.kernel,
        mesh=mesh,
        compiler_params=pltpu.CompilerParams(needs_layout_passes=False),
        out_type=jax.ShapeDtypeStruct((P * OUT_W,), jnp.float32),
        scratch_types=[
            pltpu.VMEM((CHUNK * IN_W,), jnp.float32),
            pltpu.VMEM((CHUNK * OUT_W,), jnp.float32),
            pltpu.VMEM((100 * DIM,), jnp.float32),
            pltpu.VMEM((500 * ND,), jnp.float32),
        ],
    )(_sc_kernel)
    out_flat = run(
        inputs.reshape(-1),
        atom_table.reshape(-1),
        num_table.reshape(-1),
    )
    return out_flat.reshape(B, N, OUT_W)
